# baseline (speedup 1.0000x reference)
import functools

import jax
import jax.numpy as jnp
from jax import lax
from jax.experimental import pallas as pl
from jax.experimental.pallas import tpu as pltpu
from jax.experimental.pallas import tpu_sc as plsc

LANES = 16


def _xlane_rotate(v, perm):
    dnums = lax.GatherDimensionNumbers(
        offset_dims=(), collapsed_slice_dims=(0,), start_index_map=(0,))
    return lax.gather(v, perm[:, None], dnums, (1,),
                      mode=lax.GatherScatterMode.PROMISE_IN_BOUNDS)


@functools.partial(jax.jit,
                   static_argnames=("rows_cov", "cols", "k_out", "chunk"))
def _sc_pool(x2, idxf, maskf, *, rows_cov, cols, k_out, chunk):
    info = plsc.get_sparse_core_info()
    num_workers = info.num_cores * info.num_subcores
    rows_per_w = rows_cov // num_workers
    nchunks = rows_per_w // chunk
    nvec = cols // LANES
    oflat_per_w = rows_per_w * k_out // 128

    mesh = plsc.VectorSubcoreMesh(core_axis_name="c", subcore_axis_name="s")

    @functools.partial(
        pl.kernel,
        out_type=jax.ShapeDtypeStruct((rows_cov * k_out // 128, 128),
                                      jnp.float32),
        mesh=mesh,
        compiler_params=pltpu.CompilerParams(use_tc_tiling_on_sc=True,
                                             needs_layout_passes=False),
        scratch_types=[
            pltpu.VMEM((cols,), jnp.int32),
            pltpu.VMEM((cols,), jnp.float32),
            pltpu.VMEM((cols // 128 * chunk, 128), jnp.float32),
            pltpu.VMEM((cols // 128 * chunk, 128), jnp.float32),
            pltpu.VMEM((oflat_per_w, 128), jnp.float32),
            pltpu.SemaphoreType.DMA,
            pltpu.SemaphoreType.DMA,
        ],
    )
    def k(x_hbm, idx_hbm, mask_hbm, out_hbm,
          idx_v, mask_v, xbuf0, xbuf1, oslab, sem0, sem1):
        wid = lax.axis_index("s") * info.num_cores + lax.axis_index("c")
        base = wid * rows_per_w
        pltpu.sync_copy(idx_hbm, idx_v)
        pltpu.sync_copy(mask_hbm, mask_v)
        nblk = cols // 128
        mask_regs = [mask_v[pl.ds(LANES * j, LANES)] for j in range(nvec)]
        q_regs = []
        l_regs = []
        for j in range(nvec):
            cidx = idx_v[pl.ds(LANES * j, LANES)]
            q_regs.append(
                lax.shift_right_logical(cidx, 7)
                * jnp.full((LANES,), chunk, jnp.int32))
            l_regs.append(
                lax.bitwise_and(cidx, jnp.full((LANES,), 127, jnp.int32)))
        lanes = lax.iota(jnp.int32, LANES)
        perm8 = lax.rem(lanes + 8, jnp.full((LANES,), 16, jnp.int32))
        perm4 = lax.rem(lanes + 4, jnp.full((LANES,), 16, jnp.int32))
        store_mask = lanes < k_out

        xbufs = (xbuf0, xbuf1)
        sems = (sem0, sem1)

        def start_chunk(cc, buf, sem):
            for cb in range(nblk):
                pltpu.async_copy(
                    x_hbm.at[pl.ds(base + cc * chunk, chunk),
                             pl.ds(cb * 128, 128)],
                    buf.at[pl.ds(cb * chunk, chunk)], sem)

        def wait_chunk(buf, sem):
            for cb in range(nblk):
                pltpu.make_async_copy(
                    x_hbm.at[pl.ds(base, chunk), pl.ds(0, 128)],
                    buf.at[pl.ds(cb * chunk, chunk)], sem).wait()

        start_chunk(0, xbuf0, sem0)
        start_chunk(1, xbuf1, sem1)

        @pl.loop(0, nchunks, step=2)
        def outer(c):
            for b in range(2):
                cur = c + b
                wait_chunk(xbufs[b], sems[b])
                obase = cur * chunk

                @plsc.parallel_loop(0, chunk, unroll=6)
                def row_body(r):
                    rsplat = jnp.full((LANES,), r, jnp.int32)
                    vals = [
                        plsc.load_gather(
                            xbufs[b], [rsplat + q_regs[j], l_regs[j]])
                        * mask_regs[j]
                        for j in range(nvec)
                    ]
                    while len(vals) > 1:
                        vals = [jnp.maximum(vals[2 * i], vals[2 * i + 1])
                                for i in range(len(vals) // 2)]
                    acc = vals[0]
                    acc = jnp.maximum(acc, _xlane_rotate(acc, perm8))
                    acc = jnp.maximum(acc, _xlane_rotate(acc, perm4))
                    f_loc = obase + r
                    srow = jnp.full((LANES,), (f_loc // 128) * k_out,
                                    jnp.int32) + lanes
                    scol = jnp.full((LANES,), f_loc % 128, jnp.int32)
                    plsc.store_scatter(oslab, [srow, scol], acc,
                                       mask=store_mask)

                @pl.when(cur + 2 < nchunks)
                def _():
                    start_chunk(cur + 2, xbufs[b], sems[b])

        pltpu.sync_copy(oslab, out_hbm.at[pl.ds(wid * oflat_per_w,
                                                oflat_per_w)])

    return k(x2, idxf, maskf)


RB = 1024


@functools.partial(jax.jit,
                   static_argnames=("row0", "rows_cov", "cols", "k_out"))
def _tc_pool(x2, idxf, maskf, *, row0, rows_cov, cols, k_out):
    nb = rows_cov // RB

    def body(idx_ref, mask_ref, x_ref, o_ref, p_ref):
        @pl.when(pl.program_id(0) == 0)
        def _():
            src = lax.broadcasted_iota(jnp.int32, (cols, cols), 0)
            tgt = jnp.broadcast_to(idx_ref[0, :][None, :], (cols, cols))
            msk = jnp.broadcast_to(mask_ref[0, :][None, :], (cols, cols))
            p_ref[...] = jnp.where(src == tgt, msk, 0.0)

        col = lax.dot_general(x_ref[...], p_ref[...],
                              (((1,), (0,)), ((), ())),
                              precision=lax.Precision.DEFAULT,
                              preferred_element_type=jnp.float32)
        w = cols
        while w > k_out:
            w //= 2
            col = jnp.maximum(col[:, :w], col[:, w:2 * w])
        o_ref[...] = (col.reshape(RB // 128, 128, k_out)
                      .transpose(0, 2, 1)
                      .reshape(RB * k_out // 128, 128))

    return pl.pallas_call(
        body,
        grid=(nb,),
        in_specs=[
            pl.BlockSpec((1, cols), lambda i: (0, 0)),
            pl.BlockSpec((1, cols), lambda i: (0, 0)),
            pl.BlockSpec((RB, cols), lambda i: (row0 // RB + i, 0)),
        ],
        out_specs=pl.BlockSpec((RB * k_out // 128, 128), lambda i: (i, 0)),
        out_shape=jax.ShapeDtypeStruct((rows_cov * k_out // 128, 128),
                                       jnp.float32),
        scratch_shapes=[pltpu.VMEM((cols, cols), jnp.float32)],
    )(idxf.reshape(1, cols), maskf.reshape(1, cols), x2)


SC_BATCHES = 20


def kernel(input_images, indices, mask):
    b, f, c = input_images.shape
    l, k_out = indices.shape
    rows = b * f
    x2 = input_images.reshape(rows, c)
    idxf = indices.reshape(l * k_out).astype(jnp.int32)
    maskf = mask.reshape(l * k_out).astype(jnp.float32)
    rows_sc = SC_BATCHES * f
    out_sc = _sc_pool(x2, idxf, maskf, rows_cov=rows_sc, cols=l * k_out,
                      k_out=k_out, chunk=128)
    out_tc = _tc_pool(x2, idxf, maskf, row0=rows_sc, rows_cov=rows - rows_sc,
                      cols=l * k_out, k_out=k_out)
    out2 = jnp.concatenate([out_sc, out_tc], axis=0)
    out4 = out2.reshape(b, f // 128, k_out, 128)
    return out4.transpose(0, 1, 3, 2).reshape(b, f, k_out)


# speedup vs baseline: 1.0887x; 1.0887x over previous
import functools

import jax
import jax.numpy as jnp
from jax import lax
from jax.experimental import pallas as pl
from jax.experimental.pallas import tpu as pltpu
from jax.experimental.pallas import tpu_sc as plsc

LANES = 16


def _xlane_rotate(v, perm):
    dnums = lax.GatherDimensionNumbers(
        offset_dims=(), collapsed_slice_dims=(0,), start_index_map=(0,))
    return lax.gather(v, perm[:, None], dnums, (1,),
                      mode=lax.GatherScatterMode.PROMISE_IN_BOUNDS)


@functools.partial(jax.jit,
                   static_argnames=("rows_cov", "cols", "k_out", "chunk"))
def _sc_pool(x2, idxf, maskf, *, rows_cov, cols, k_out, chunk):
    info = plsc.get_sparse_core_info()
    num_workers = info.num_cores * info.num_subcores
    rows_per_w = rows_cov // num_workers
    nchunks = rows_per_w // chunk
    nvec = cols // LANES
    oflat_per_w = rows_per_w * k_out // 128

    mesh = plsc.VectorSubcoreMesh(core_axis_name="c", subcore_axis_name="s")

    @functools.partial(
        pl.kernel,
        out_type=jax.ShapeDtypeStruct((rows_cov * k_out // 128, 128),
                                      jnp.float32),
        mesh=mesh,
        compiler_params=pltpu.CompilerParams(use_tc_tiling_on_sc=True,
                                             needs_layout_passes=False),
        scratch_types=[
            pltpu.VMEM((cols,), jnp.int32),
            pltpu.VMEM((cols,), jnp.float32),
            pltpu.VMEM((cols // 128 * chunk, 128), jnp.float32),
            pltpu.VMEM((cols // 128 * chunk, 128), jnp.float32),
            pltpu.VMEM((oflat_per_w, 128), jnp.float32),
            pltpu.SemaphoreType.DMA,
            pltpu.SemaphoreType.DMA,
        ],
    )
    def k(x_hbm, idx_hbm, mask_hbm, out_hbm,
          idx_v, mask_v, xbuf0, xbuf1, oslab, sem0, sem1):
        wid = lax.axis_index("s") * info.num_cores + lax.axis_index("c")
        base = wid * rows_per_w
        pltpu.sync_copy(idx_hbm, idx_v)
        pltpu.sync_copy(mask_hbm, mask_v)
        nblk = cols // 128
        mask_regs = [mask_v[pl.ds(LANES * j, LANES)] for j in range(nvec)]
        q_regs = []
        l_regs = []
        for j in range(nvec):
            cidx = idx_v[pl.ds(LANES * j, LANES)]
            q_regs.append(
                lax.shift_right_logical(cidx, 7)
                * jnp.full((LANES,), chunk, jnp.int32))
            l_regs.append(
                lax.bitwise_and(cidx, jnp.full((LANES,), 127, jnp.int32)))
        lanes = lax.iota(jnp.int32, LANES)
        perm8 = lax.rem(lanes + 8, jnp.full((LANES,), 16, jnp.int32))
        perm4 = lax.rem(lanes + 4, jnp.full((LANES,), 16, jnp.int32))
        store_mask = lanes < k_out

        xbufs = (xbuf0, xbuf1)
        sems = (sem0, sem1)

        def start_chunk(cc, buf, sem):
            for cb in range(nblk):
                pltpu.async_copy(
                    x_hbm.at[pl.ds(base + cc * chunk, chunk),
                             pl.ds(cb * 128, 128)],
                    buf.at[pl.ds(cb * chunk, chunk)], sem)

        def wait_chunk(buf, sem):
            for cb in range(nblk):
                pltpu.make_async_copy(
                    x_hbm.at[pl.ds(base, chunk), pl.ds(0, 128)],
                    buf.at[pl.ds(cb * chunk, chunk)], sem).wait()

        start_chunk(0, xbuf0, sem0)
        start_chunk(1, xbuf1, sem1)

        @pl.loop(0, nchunks, step=2)
        def outer(c):
            for b in range(2):
                cur = c + b
                wait_chunk(xbufs[b], sems[b])
                obase = cur * chunk

                @plsc.parallel_loop(0, chunk, unroll=4)
                def row_body(r):
                    rsplat = jnp.full((LANES,), r, jnp.int32)
                    vals = [
                        plsc.load_gather(
                            xbufs[b], [rsplat + q_regs[j], l_regs[j]])
                        * mask_regs[j]
                        for j in range(nvec)
                    ]
                    while len(vals) > 1:
                        vals = [jnp.maximum(vals[2 * i], vals[2 * i + 1])
                                for i in range(len(vals) // 2)]
                    acc = vals[0]
                    acc = jnp.maximum(acc, _xlane_rotate(acc, perm8))
                    acc = jnp.maximum(acc, _xlane_rotate(acc, perm4))
                    f_loc = obase + r
                    srow = jnp.full((LANES,), (f_loc // 128) * k_out,
                                    jnp.int32) + lanes
                    scol = jnp.full((LANES,), f_loc % 128, jnp.int32)
                    plsc.store_scatter(oslab, [srow, scol], acc,
                                       mask=store_mask)

                @pl.when(cur + 2 < nchunks)
                def _():
                    start_chunk(cur + 2, xbufs[b], sems[b])

        pltpu.sync_copy(oslab, out_hbm.at[pl.ds(wid * oflat_per_w,
                                                oflat_per_w)])

    return k(x2, idxf, maskf)


RB = 1024


@functools.partial(jax.jit,
                   static_argnames=("row0", "rows_cov", "cols", "k_out"))
def _tc_pool(x2, idxf, maskf, *, row0, rows_cov, cols, k_out):
    nb = rows_cov // RB

    def body(idx_ref, mask_ref, x_ref, o_ref, p_ref):
        @pl.when(pl.program_id(0) == 0)
        def _():
            src = lax.broadcasted_iota(jnp.int32, (cols, cols), 0)
            tgt = jnp.broadcast_to(idx_ref[0, :][None, :], (cols, cols))
            msk = jnp.broadcast_to(mask_ref[0, :][None, :], (cols, cols))
            p_ref[...] = jnp.where(src == tgt, msk, 0.0)

        col = lax.dot_general(x_ref[...], p_ref[...],
                              (((1,), (0,)), ((), ())),
                              precision=lax.Precision.DEFAULT,
                              preferred_element_type=jnp.float32)
        w = cols
        while w > k_out:
            w //= 2
            col = jnp.maximum(col[:, :w], col[:, w:2 * w])
        o_ref[...] = (col.reshape(RB // 128, 128, k_out)
                      .transpose(0, 2, 1)
                      .reshape(RB * k_out // 128, 128))

    return pl.pallas_call(
        body,
        grid=(nb,),
        in_specs=[
            pl.BlockSpec((1, cols), lambda i: (0, 0)),
            pl.BlockSpec((1, cols), lambda i: (0, 0)),
            pl.BlockSpec((RB, cols), lambda i: (row0 // RB + i, 0)),
        ],
        out_specs=pl.BlockSpec((RB * k_out // 128, 128), lambda i: (i, 0)),
        out_shape=jax.ShapeDtypeStruct((rows_cov * k_out // 128, 128),
                                       jnp.float32),
        scratch_shapes=[pltpu.VMEM((cols, cols), jnp.float32)],
    )(idxf.reshape(1, cols), maskf.reshape(1, cols), x2)


SC_BATCHES = 20


def kernel(input_images, indices, mask):
    b, f, c = input_images.shape
    l, k_out = indices.shape
    rows = b * f
    x2 = input_images.reshape(rows, c)
    idxf = indices.reshape(l * k_out).astype(jnp.int32)
    maskf = mask.reshape(l * k_out).astype(jnp.float32)
    rows_sc = SC_BATCHES * f
    out_sc = _sc_pool(x2, idxf, maskf, rows_cov=rows_sc, cols=l * k_out,
                      k_out=k_out, chunk=128)
    out_tc = _tc_pool(x2, idxf, maskf, row0=rows_sc, rows_cov=rows - rows_sc,
                      cols=l * k_out, k_out=k_out)
    out2 = jnp.concatenate([out_sc, out_tc], axis=0)
    out4 = out2.reshape(b, f // 128, k_out, 128)
    return out4.transpose(0, 1, 3, 2).reshape(b, f, k_out)
